# baseline (speedup 1.0000x reference)
import functools

import jax
import jax.numpy as jnp
from jax import lax
from jax.experimental import pallas as pl
from jax.experimental.pallas import tpu as pltpu
from jax.experimental.pallas import tpu_sc as plsc

N_NODES = 10000
N_EDGES = 320000
E_TOT = N_EDGES + N_NODES
D_FEAT = 128
GDIM = 64
N_GRAPHS = 256
EPS = 1e-5

NC, NS = 2, 16
NW = NC * NS
CHUNK = 128
CPW = (-(-E_TOT // (NW * CHUNK)) + 3) // 4 * 4
E_PAD = NW * CPW * CHUNK
TOTCH = NW * CPW + 4
ACCW = 80
RCH = 80
NRC = N_NODES // RCH
KPS = -(-NRC // NS)


def _sc_edge_body(hl_hbm, hr_hbm, idx_hbm, att_hbm, out_hbm,
                  i0, i1, i2, i3, A0, A1, B0, B1, S0, S1, att_v, zbuf, acc_sh,
                  si0, si1, si2, si3, sg0, sg1, ss0, ss1):
    c = lax.axis_index("c")
    s = lax.axis_index("s")
    wid = s * NC + c
    IDX = (i0, i1, i2, i3)
    SI = (si0, si1, si2, si3)
    A = (A0, A1)
    B = (B0, B1)
    S = (S0, S1)
    SG = (sg0, sg1)
    SS = (ss0, ss1)

    zero16 = jnp.zeros((16,), jnp.float32)

    def zrow(i, carry):
        for q in range(ACCW // 16):
            zbuf[i, pl.ds(q * 16, 16)] = zero16
        return carry

    lax.fori_loop(0, RCH, zrow, 0)

    def zchunk(k, carry):
        cid = s + k * NS

        @pl.when(cid < NRC)
        def _():
            pltpu.sync_copy(zbuf, acc_sh.at[pl.ds(cid * RCH, RCH)])

        return carry

    lax.fori_loop(0, KPS, zchunk, 0)
    pltpu.sync_copy(att_hbm, att_v)

    def szero(i, carry):
        for q in range(ACCW // 16):
            S0[i, pl.ds(q * 16, 16)] = zero16
            S1[i, pl.ds(q * 16, 16)] = zero16
        return carry

    lax.fori_loop(0, CHUNK, szero, 0)
    plsc.subcore_barrier()

    iota16 = lax.iota(jnp.int32, 16)

    def compute(cid, Ar, Br, Sr):
        base = cid * CHUNK

        def grp(g, carry2):
            rows = g * 16 + iota16

            def fstep(f, eacc):
                colv = jnp.full((16,), f, jnp.int32)
                av = plsc.load_gather(Ar, [rows, colv])
                bv = plsc.load_gather(Br, [rows, colv])
                t = av + bv
                t = jnp.maximum(t, 0.2 * t)
                ti = plsc.bitcast(t, jnp.int32)
                ti = ti + 0x7FFF + ((ti >> 16) & 1)
                ti = ti & jnp.int32(-65536)
                t = plsc.bitcast(ti, jnp.float32)
                attf = att_v[f, pl.ds(0, 16)]
                return eacc + t * attf

            e = lax.fori_loop(0, GDIM, fstep, jnp.zeros((16,), jnp.float32),
                              unroll=8)
            gid = base + g * 16 + iota16
            w = jnp.exp(e) * (gid < E_TOT).astype(jnp.float32)

            def f2(f, carry3):
                colv = jnp.full((16,), f, jnp.int32)
                av = plsc.load_gather(Ar, [rows, colv])
                plsc.store_scatter(Sr, [rows, colv], av * w)
                return carry3

            lax.fori_loop(0, GDIM, f2, 0, unroll=8)
            plsc.store_scatter(Sr, [rows, jnp.full((16,), GDIM, jnp.int32)], w)
            return carry2

        lax.fori_loop(0, CHUNK // 16, grp, 0)

    base_ch = wid * CPW
    for t in range(3):
        pltpu.async_copy(idx_hbm.at[base_ch + t], IDX[t], SI[t])
    pltpu.make_async_copy(idx_hbm.at[base_ch], IDX[0], SI[0]).wait()
    pltpu.async_copy(hl_hbm.at[IDX[0].at[0]], A[0], SG[0])
    pltpu.async_copy(hr_hbm.at[IDX[0].at[1]], B[0], SG[0])

    def step_body(step, carry):
        for jj in range(4):
            cid = base_ch + step * 4 + jj
            nj = (jj + 1) % 4
            pj = (jj + 3) % 4
            g = jj % 2
            ng = (jj + 1) % 2
            pltpu.make_async_copy(idx_hbm.at[cid + 1], IDX[nj], SI[nj]).wait()
            pltpu.async_copy(hl_hbm.at[IDX[nj].at[0]], A[ng], SG[ng])
            pltpu.async_copy(hr_hbm.at[IDX[nj].at[1]], B[ng], SG[ng])
            pltpu.make_async_copy(hl_hbm.at[IDX[jj].at[0]], A[g], SG[g]).wait()
            pltpu.make_async_copy(hr_hbm.at[IDX[jj].at[1]], B[g], SG[g]).wait()

            def w3():
                pltpu.make_async_copy(
                    S[ng], acc_sh.at[IDX[pj].at[1]], SS[ng]).wait()

            if jj == 0:
                pl.when(step > 0)(w3)
            else:
                w3()
            pltpu.async_copy(idx_hbm.at[cid + 3], IDX[pj], SI[pj])
            compute(cid, A[g], B[g], S[g])
            pltpu.async_copy(S[g], acc_sh.at[IDX[jj].at[1]], SS[g], add=True)
        return carry

    lax.fori_loop(0, CPW // 4, step_body, 0)
    pltpu.make_async_copy(hl_hbm.at[IDX[0].at[0]], A[0], SG[0]).wait()
    pltpu.make_async_copy(hr_hbm.at[IDX[0].at[1]], B[0], SG[0]).wait()
    pltpu.make_async_copy(idx_hbm.at[0], IDX[1], SI[1]).wait()
    pltpu.make_async_copy(idx_hbm.at[0], IDX[2], SI[2]).wait()
    pltpu.make_async_copy(S[1], acc_sh.at[IDX[3].at[1]], SS[1]).wait()
    plsc.subcore_barrier()

    def dchunk(k, carry):
        cid = s + k * NS

        @pl.when(cid < NRC)
        def _():
            sl = pl.ds(cid * RCH, RCH)
            pltpu.sync_copy(acc_sh.at[sl], out_hbm.at[c, sl])

        return carry

    lax.fori_loop(0, KPS, dchunk, 0)


def _sc_edge(hl, hr, idx_all, att):
    att = lax.reduce_precision(att, exponent_bits=8, mantissa_bits=7)
    att = jnp.broadcast_to(att[:, None], (GDIM, 16))
    mesh = plsc.VectorSubcoreMesh(core_axis_name="c", subcore_axis_name="s")
    fn = pl.kernel(
        _sc_edge_body,
        mesh=mesh,
        out_type=jax.ShapeDtypeStruct((NC, N_NODES, ACCW), jnp.float32),
        scratch_types=(
            [pltpu.VMEM((2, CHUNK), jnp.int32)] * 4 +
            [pltpu.VMEM((CHUNK, GDIM), jnp.float32)] * 4 +
            [pltpu.VMEM((CHUNK, ACCW), jnp.float32)] * 2 +
            [pltpu.VMEM((GDIM, 16), jnp.float32),
             pltpu.VMEM((RCH, ACCW), jnp.float32),
             pltpu.VMEM_SHARED((N_NODES, ACCW), jnp.float32)] +
            [pltpu.SemaphoreType.DMA] * 8
        ),
        compiler_params=pltpu.CompilerParams(
            needs_layout_passes=False, use_tc_tiling_on_sc=False),
    )
    return fn(hl, hr, idx_all, att)


def _tc_in_body(x_ref, wl_ref, wr_ref, hl_ref, hr_ref):
    x = x_ref[...]
    hl_ref[...] = jnp.dot(x, wl_ref[...], preferred_element_type=jnp.float32)
    hr_ref[...] = jnp.dot(x, wr_ref[...], preferred_element_type=jnp.float32)


def _finalize_h(acc_ref, bias_ref, gam_ref, bet_ref):
    a0 = acc_ref[0]
    a1 = acc_ref[1]
    num = a0[:, :GDIM] + a1[:, :GDIM]
    den = a0[:, GDIM:GDIM + 1] + a1[:, GDIM:GDIM + 1]
    h = num / den + bias_ref[...]
    h = jnp.maximum(h, 0.0)
    mu = jnp.mean(h, axis=0, keepdims=True)
    var = jnp.mean((h - mu) * (h - mu), axis=0, keepdims=True)
    return gam_ref[...] * (h - mu) / jnp.sqrt(var + EPS) + bet_ref[...]


def _tc_mid_body(acc_ref, bias_ref, gam_ref, bet_ref, wl_ref, wr_ref,
                 hl_ref, hr_ref):
    hbn = _finalize_h(acc_ref, bias_ref, gam_ref, bet_ref)
    hl_ref[...] = jnp.dot(hbn, wl_ref[...], preferred_element_type=jnp.float32)
    hr_ref[...] = jnp.dot(hbn, wr_ref[...], preferred_element_type=jnp.float32)


def _tc_out_body(acc_ref, bias_ref, gam_ref, bet_ref, batch_ref,
                 w1, b1, g1, be1, w2, b2, g2, be2, w3, b3, g3, be3,
                 wo, bo, out_ref):
    hbn = _finalize_h(acc_ref, bias_ref, gam_ref, bet_ref)
    gids = lax.broadcasted_iota(jnp.int32, (N_GRAPHS, N_NODES), 0)
    onehot = (gids == batch_ref[...]).astype(jnp.float32)
    g = jnp.dot(onehot, hbn, preferred_element_type=jnp.float32,
                precision=lax.Precision.HIGHEST)
    for wr, br, gr, ber in ((w1, b1, g1, be1), (w2, b2, g2, be2),
                            (w3, b3, g3, be3)):
        g = jnp.dot(g, wr[...], preferred_element_type=jnp.float32) + br[...]
        g = jnp.maximum(g, 0.0)
        mu = jnp.mean(g, axis=0, keepdims=True)
        var = jnp.mean((g - mu) * (g - mu), axis=0, keepdims=True)
        g = gr[...] * (g - mu) / jnp.sqrt(var + EPS) + ber[...]
    out_ref[...] = jnp.dot(g, wo[...], preferred_element_type=jnp.float32) + bo[...]


def _r2(v):
    return v.reshape(1, -1)


def kernel(x, edge_index, batch, params):
    loop = jnp.arange(N_NODES, dtype=edge_index.dtype)
    pad = jnp.zeros((E_PAD - E_TOT,), edge_index.dtype)
    src = jnp.concatenate([edge_index[0], loop, pad]).reshape(-1, CHUNK)
    dst = jnp.concatenate([edge_index[1], loop, pad]).reshape(-1, CHUNK)
    idx_all = jnp.concatenate(
        [jnp.stack([src, dst], axis=1),
         jnp.zeros((TOTCH - NW * CPW, 2, CHUNK), edge_index.dtype)])

    conv = params['conv']
    cbn = params['conv_bn']

    hl, hr = pl.pallas_call(
        _tc_in_body,
        out_shape=[jax.ShapeDtypeStruct((N_NODES, GDIM), jnp.float32)] * 2,
    )(x, conv[0]['Wl'], conv[0]['Wr'])

    for l in range(2):
        acc = _sc_edge(hl, hr, idx_all, conv[l]['att'])
        hl, hr = pl.pallas_call(
            _tc_mid_body,
            out_shape=[jax.ShapeDtypeStruct((N_NODES, GDIM), jnp.float32)] * 2,
        )(acc, _r2(conv[l]['bias']), _r2(cbn[l]['gamma']), _r2(cbn[l]['beta']),
          conv[l + 1]['Wl'], conv[l + 1]['Wr'])

    acc = _sc_edge(hl, hr, idx_all, conv[2]['att'])
    dense = params['dense']
    dbn = params['dense_bn']
    dargs = []
    for l in range(3):
        dargs += [dense[l]['W'], _r2(dense[l]['b']),
                  _r2(dbn[l]['gamma']), _r2(dbn[l]['beta'])]
    y = pl.pallas_call(
        _tc_out_body,
        out_shape=jax.ShapeDtypeStruct((N_GRAPHS, 1), jnp.float32),
    )(acc, _r2(conv[2]['bias']), _r2(cbn[2]['gamma']), _r2(cbn[2]['beta']),
      batch.reshape(1, -1), *dargs, params['out']['W'], _r2(params['out']['b']))
    return y


# speedup vs baseline: 1.0187x; 1.0187x over previous
import functools

import jax
import jax.numpy as jnp
from jax import lax
from jax.experimental import pallas as pl
from jax.experimental.pallas import tpu as pltpu
from jax.experimental.pallas import tpu_sc as plsc

N_NODES = 10000
N_EDGES = 320000
E_TOT = N_EDGES + N_NODES
D_FEAT = 128
GDIM = 64
N_GRAPHS = 256
EPS = 1e-5

NC, NS = 2, 16
NW = NC * NS
CHUNK = 128
CPW = (-(-E_TOT // (NW * CHUNK)) + 3) // 4 * 4
E_PAD = NW * CPW * CHUNK
TOTCH = NW * CPW + 4
ACCW = 80
RCH = 80
NRC = N_NODES // RCH
KPS = -(-NRC // NS)


def _sc_edge_body(hl_hbm, hr_hbm, idx_hbm, att_hbm, out_hbm,
                  i0, i1, i2, i3, A0, A1, B0, B1, S0, S1, att_v, zbuf, acc_sh,
                  si0, si1, si2, si3, sg0, sg1, ss0, ss1):
    c = lax.axis_index("c")
    s = lax.axis_index("s")
    wid = s * NC + c
    IDX = (i0, i1, i2, i3)
    SI = (si0, si1, si2, si3)
    A = (A0, A1)
    B = (B0, B1)
    S = (S0, S1)
    SG = (sg0, sg1)
    SS = (ss0, ss1)

    zero16 = jnp.zeros((16,), jnp.float32)

    def zrow(i, carry):
        for q in range(ACCW // 16):
            zbuf[i, pl.ds(q * 16, 16)] = zero16
        return carry

    lax.fori_loop(0, RCH, zrow, 0)

    def zchunk(k, carry):
        cid = s + k * NS

        @pl.when(cid < NRC)
        def _():
            pltpu.sync_copy(zbuf, acc_sh.at[pl.ds(cid * RCH, RCH)])

        return carry

    lax.fori_loop(0, KPS, zchunk, 0)
    pltpu.sync_copy(att_hbm, att_v)

    def szero(i, carry):
        for q in range(ACCW // 16):
            S0[i, pl.ds(q * 16, 16)] = zero16
            S1[i, pl.ds(q * 16, 16)] = zero16
        return carry

    lax.fori_loop(0, CHUNK, szero, 0)
    plsc.subcore_barrier()

    iota16 = lax.iota(jnp.int32, 16)

    def compute(cid, Ar, Br, Sr):
        base = cid * CHUNK

        def grp(g, carry2):
            rows = g * 16 + iota16

            def fstep(f, eacc):
                colv = jnp.full((16,), f, jnp.int32)
                av = plsc.load_gather(Ar, [rows, colv])
                bv = plsc.load_gather(Br, [rows, colv])
                t = av + bv
                t = jnp.maximum(t, 0.2 * t)
                ti = plsc.bitcast(t, jnp.int32)
                ti = ti + 0x7FFF + ((ti >> 16) & 1)
                ti = ti & jnp.int32(-65536)
                t = plsc.bitcast(ti, jnp.float32)
                attf = att_v[f, pl.ds(0, 16)]
                return eacc + t * attf

            e = lax.fori_loop(0, GDIM, fstep, jnp.zeros((16,), jnp.float32))
            gid = base + g * 16 + iota16
            w = jnp.exp(e) * (gid < E_TOT).astype(jnp.float32)

            def f2(f, carry3):
                colv = jnp.full((16,), f, jnp.int32)
                av = plsc.load_gather(Ar, [rows, colv])
                plsc.store_scatter(Sr, [rows, colv], av * w)
                return carry3

            lax.fori_loop(0, GDIM, f2, 0)
            plsc.store_scatter(Sr, [rows, jnp.full((16,), GDIM, jnp.int32)], w)
            return carry2

        lax.fori_loop(0, CHUNK // 16, grp, 0)

    base_ch = wid * CPW
    for t in range(3):
        pltpu.async_copy(idx_hbm.at[base_ch + t], IDX[t], SI[t])
    pltpu.make_async_copy(idx_hbm.at[base_ch], IDX[0], SI[0]).wait()
    for q4 in range(4):
        sl4 = pl.ds(q4 * 32, 32)
        pltpu.async_copy(hl_hbm.at[IDX[0].at[0, sl4]], A[0].at[sl4], SG[0])
        pltpu.async_copy(hr_hbm.at[IDX[0].at[1, sl4]], B[0].at[sl4], SG[0])

    def step_body(step, carry):
        for jj in range(4):
            cid = base_ch + step * 4 + jj
            nj = (jj + 1) % 4
            pj = (jj + 3) % 4
            g = jj % 2
            ng = (jj + 1) % 2
            pltpu.make_async_copy(idx_hbm.at[cid + 1], IDX[nj], SI[nj]).wait()
            for q4 in range(4):
                sl4 = pl.ds(q4 * 32, 32)
                pltpu.async_copy(hl_hbm.at[IDX[nj].at[0, sl4]],
                                 A[ng].at[sl4], SG[ng])
                pltpu.async_copy(hr_hbm.at[IDX[nj].at[1, sl4]],
                                 B[ng].at[sl4], SG[ng])
            for q4 in range(4):
                sl4 = pl.ds(q4 * 32, 32)
                pltpu.make_async_copy(hl_hbm.at[IDX[jj].at[0, sl4]],
                                      A[g].at[sl4], SG[g]).wait()
                pltpu.make_async_copy(hr_hbm.at[IDX[jj].at[1, sl4]],
                                      B[g].at[sl4], SG[g]).wait()

            def w3():
                pltpu.make_async_copy(
                    S[ng], acc_sh.at[IDX[pj].at[1]], SS[ng]).wait()

            if jj == 0:
                pl.when(step > 0)(w3)
            else:
                w3()
            pltpu.async_copy(idx_hbm.at[cid + 3], IDX[pj], SI[pj])
            compute(cid, A[g], B[g], S[g])
            pltpu.async_copy(S[g], acc_sh.at[IDX[jj].at[1]], SS[g], add=True)
        return carry

    lax.fori_loop(0, CPW // 4, step_body, 0)
    for q4 in range(4):
        sl4 = pl.ds(q4 * 32, 32)
        pltpu.make_async_copy(hl_hbm.at[IDX[0].at[0, sl4]],
                              A[0].at[sl4], SG[0]).wait()
        pltpu.make_async_copy(hr_hbm.at[IDX[0].at[1, sl4]],
                              B[0].at[sl4], SG[0]).wait()
    pltpu.make_async_copy(idx_hbm.at[0], IDX[1], SI[1]).wait()
    pltpu.make_async_copy(idx_hbm.at[0], IDX[2], SI[2]).wait()
    pltpu.make_async_copy(S[1], acc_sh.at[IDX[3].at[1]], SS[1]).wait()
    plsc.subcore_barrier()

    def dchunk(k, carry):
        cid = s + k * NS

        @pl.when(cid < NRC)
        def _():
            sl = pl.ds(cid * RCH, RCH)
            pltpu.sync_copy(acc_sh.at[sl], out_hbm.at[c, sl])

        return carry

    lax.fori_loop(0, KPS, dchunk, 0)


def _sc_edge(hl, hr, idx_all, att):
    att = lax.reduce_precision(att, exponent_bits=8, mantissa_bits=7)
    att = jnp.broadcast_to(att[:, None], (GDIM, 16))
    mesh = plsc.VectorSubcoreMesh(core_axis_name="c", subcore_axis_name="s")
    fn = pl.kernel(
        _sc_edge_body,
        mesh=mesh,
        out_type=jax.ShapeDtypeStruct((NC, N_NODES, ACCW), jnp.float32),
        scratch_types=(
            [pltpu.VMEM((2, CHUNK), jnp.int32)] * 4 +
            [pltpu.VMEM((CHUNK, GDIM), jnp.float32)] * 4 +
            [pltpu.VMEM((CHUNK, ACCW), jnp.float32)] * 2 +
            [pltpu.VMEM((GDIM, 16), jnp.float32),
             pltpu.VMEM((RCH, ACCW), jnp.float32),
             pltpu.VMEM_SHARED((N_NODES, ACCW), jnp.float32)] +
            [pltpu.SemaphoreType.DMA] * 8
        ),
        compiler_params=pltpu.CompilerParams(
            needs_layout_passes=False, use_tc_tiling_on_sc=False),
    )
    return fn(hl, hr, idx_all, att)


def _tc_in_body(x_ref, wl_ref, wr_ref, hl_ref, hr_ref):
    x = x_ref[...]
    hl_ref[...] = jnp.dot(x, wl_ref[...], preferred_element_type=jnp.float32)
    hr_ref[...] = jnp.dot(x, wr_ref[...], preferred_element_type=jnp.float32)


def _finalize_h(acc_ref, bias_ref, gam_ref, bet_ref):
    a0 = acc_ref[0]
    a1 = acc_ref[1]
    num = a0[:, :GDIM] + a1[:, :GDIM]
    den = a0[:, GDIM:GDIM + 1] + a1[:, GDIM:GDIM + 1]
    h = num / den + bias_ref[...]
    h = jnp.maximum(h, 0.0)
    mu = jnp.mean(h, axis=0, keepdims=True)
    var = jnp.mean((h - mu) * (h - mu), axis=0, keepdims=True)
    return gam_ref[...] * (h - mu) / jnp.sqrt(var + EPS) + bet_ref[...]


def _tc_mid_body(acc_ref, bias_ref, gam_ref, bet_ref, wl_ref, wr_ref,
                 hl_ref, hr_ref):
    hbn = _finalize_h(acc_ref, bias_ref, gam_ref, bet_ref)
    hl_ref[...] = jnp.dot(hbn, wl_ref[...], preferred_element_type=jnp.float32)
    hr_ref[...] = jnp.dot(hbn, wr_ref[...], preferred_element_type=jnp.float32)


def _tc_out_body(acc_ref, bias_ref, gam_ref, bet_ref, batch_ref,
                 w1, b1, g1, be1, w2, b2, g2, be2, w3, b3, g3, be3,
                 wo, bo, out_ref):
    hbn = _finalize_h(acc_ref, bias_ref, gam_ref, bet_ref)
    gids = lax.broadcasted_iota(jnp.int32, (N_GRAPHS, N_NODES), 0)
    onehot = (gids == batch_ref[...]).astype(jnp.float32)
    g = jnp.dot(onehot, hbn, preferred_element_type=jnp.float32,
                precision=lax.Precision.HIGHEST)
    for wr, br, gr, ber in ((w1, b1, g1, be1), (w2, b2, g2, be2),
                            (w3, b3, g3, be3)):
        g = jnp.dot(g, wr[...], preferred_element_type=jnp.float32) + br[...]
        g = jnp.maximum(g, 0.0)
        mu = jnp.mean(g, axis=0, keepdims=True)
        var = jnp.mean((g - mu) * (g - mu), axis=0, keepdims=True)
        g = gr[...] * (g - mu) / jnp.sqrt(var + EPS) + ber[...]
    out_ref[...] = jnp.dot(g, wo[...], preferred_element_type=jnp.float32) + bo[...]


def _r2(v):
    return v.reshape(1, -1)


def kernel(x, edge_index, batch, params):
    loop = jnp.arange(N_NODES, dtype=edge_index.dtype)
    pad = jnp.zeros((E_PAD - E_TOT,), edge_index.dtype)
    src = jnp.concatenate([edge_index[0], loop, pad]).reshape(-1, CHUNK)
    dst = jnp.concatenate([edge_index[1], loop, pad]).reshape(-1, CHUNK)
    idx_all = jnp.concatenate(
        [jnp.stack([src, dst], axis=1),
         jnp.zeros((TOTCH - NW * CPW, 2, CHUNK), edge_index.dtype)])

    conv = params['conv']
    cbn = params['conv_bn']

    hl, hr = pl.pallas_call(
        _tc_in_body,
        out_shape=[jax.ShapeDtypeStruct((N_NODES, GDIM), jnp.float32)] * 2,
    )(x, conv[0]['Wl'], conv[0]['Wr'])

    for l in range(2):
        acc = _sc_edge(hl, hr, idx_all, conv[l]['att'])
        hl, hr = pl.pallas_call(
            _tc_mid_body,
            out_shape=[jax.ShapeDtypeStruct((N_NODES, GDIM), jnp.float32)] * 2,
        )(acc, _r2(conv[l]['bias']), _r2(cbn[l]['gamma']), _r2(cbn[l]['beta']),
          conv[l + 1]['Wl'], conv[l + 1]['Wr'])

    acc = _sc_edge(hl, hr, idx_all, conv[2]['att'])
    dense = params['dense']
    dbn = params['dense_bn']
    dargs = []
    for l in range(3):
        dargs += [dense[l]['W'], _r2(dense[l]['b']),
                  _r2(dbn[l]['gamma']), _r2(dbn[l]['beta'])]
    y = pl.pallas_call(
        _tc_out_body,
        out_shape=jax.ShapeDtypeStruct((N_GRAPHS, 1), jnp.float32),
    )(acc, _r2(conv[2]['bias']), _r2(cbn[2]['gamma']), _r2(cbn[2]['beta']),
      batch.reshape(1, -1), *dargs, params['out']['W'], _r2(params['out']['b']))
    return y


# speedup vs baseline: 1.0191x; 1.0004x over previous
import functools

import jax
import jax.numpy as jnp
from jax import lax
from jax.experimental import pallas as pl
from jax.experimental.pallas import tpu as pltpu
from jax.experimental.pallas import tpu_sc as plsc

N_NODES = 10000
N_EDGES = 320000
E_TOT = N_EDGES + N_NODES
D_FEAT = 128
GDIM = 64
N_GRAPHS = 256
EPS = 1e-5

NC, NS = 2, 16
NW = NC * NS
CHUNK = 128
CPW = (-(-E_TOT // (NW * CHUNK)) + 3) // 4 * 4
E_PAD = NW * CPW * CHUNK
TOTCH = NW * CPW + 4
ACCW = 80
RCH = 80
NRC = N_NODES // RCH
KPS = -(-NRC // NS)


def _sc_edge_body(hl_hbm, hr_hbm, idx_hbm, att_hbm, out_hbm,
                  i0, i1, i2, i3, A0, A1, B0, B1, S0, S1, att_v, zbuf, acc_sh,
                  si0, si1, si2, si3, sg0, sg1, ss0, ss1):
    c = lax.axis_index("c")
    s = lax.axis_index("s")
    wid = s * NC + c
    IDX = (i0, i1, i2, i3)
    SI = (si0, si1, si2, si3)
    A = (A0, A1)
    B = (B0, B1)
    S = (S0, S1)
    SG = (sg0, sg1)
    SS = (ss0, ss1)

    zero16 = jnp.zeros((16,), jnp.float32)

    def zrow(i, carry):
        for q in range(ACCW // 16):
            zbuf[i, pl.ds(q * 16, 16)] = zero16
        return carry

    lax.fori_loop(0, RCH, zrow, 0)

    def zchunk(k, carry):
        cid = s + k * NS

        @pl.when(cid < NRC)
        def _():
            pltpu.sync_copy(zbuf, acc_sh.at[pl.ds(cid * RCH, RCH)])

        return carry

    lax.fori_loop(0, KPS, zchunk, 0)
    pltpu.sync_copy(att_hbm, att_v)

    def szero(i, carry):
        for q in range(ACCW // 16):
            S0[i, pl.ds(q * 16, 16)] = zero16
            S1[i, pl.ds(q * 16, 16)] = zero16
        return carry

    lax.fori_loop(0, CHUNK, szero, 0)
    plsc.subcore_barrier()

    iota16 = lax.iota(jnp.int32, 16)

    def compute(cid, Ar, Br, Sr):
        base = cid * CHUNK

        def grp(g, carry2):
            rows = g * 16 + iota16

            def fstep(f, eacc):
                colv = jnp.full((16,), f, jnp.int32)
                av = plsc.load_gather(Ar, [rows, colv])
                bv = plsc.load_gather(Br, [rows, colv])
                t = av + bv
                t = jnp.maximum(t, 0.2 * t)
                ti = plsc.bitcast(t, jnp.int32)
                ti = ti + 0x7FFF + ((ti >> 16) & 1)
                ti = ti & jnp.int32(-65536)
                t = plsc.bitcast(ti, jnp.float32)
                attf = att_v[f, pl.ds(0, 16)]
                return eacc + t * attf

            e = lax.fori_loop(0, GDIM, fstep, jnp.zeros((16,), jnp.float32))
            gid = base + g * 16 + iota16
            w = jnp.exp(e) * (gid < E_TOT).astype(jnp.float32)

            def f2(f, carry3):
                colv = jnp.full((16,), f, jnp.int32)
                av = plsc.load_gather(Ar, [rows, colv])
                plsc.store_scatter(Sr, [rows, colv], av * w)
                return carry3

            lax.fori_loop(0, GDIM, f2, 0)
            plsc.store_scatter(Sr, [rows, jnp.full((16,), GDIM, jnp.int32)], w)
            return carry2

        lax.fori_loop(0, CHUNK // 16, grp, 0)

    base_ch = wid * CPW
    for t in range(3):
        pltpu.async_copy(idx_hbm.at[base_ch + t], IDX[t], SI[t])
    pltpu.make_async_copy(idx_hbm.at[base_ch], IDX[0], SI[0]).wait()
    for q4 in range(4):
        sl4 = pl.ds(q4 * 32, 32)
        pltpu.async_copy(hl_hbm.at[IDX[0].at[0, sl4]], A[0].at[sl4], SG[0])
        pltpu.async_copy(hr_hbm.at[IDX[0].at[1, sl4]], B[0].at[sl4], SG[0])

    def step_body(step, carry):
        for jj in range(4):
            cid = base_ch + step * 4 + jj
            nj = (jj + 1) % 4
            pj = (jj + 3) % 4
            g = jj % 2
            ng = (jj + 1) % 2
            pltpu.make_async_copy(idx_hbm.at[cid + 1], IDX[nj], SI[nj]).wait()
            for q4 in range(4):
                sl4 = pl.ds(q4 * 32, 32)
                pltpu.async_copy(hl_hbm.at[IDX[nj].at[0, sl4]],
                                 A[ng].at[sl4], SG[ng])
                pltpu.async_copy(hr_hbm.at[IDX[nj].at[1, sl4]],
                                 B[ng].at[sl4], SG[ng])
            for q4 in range(4):
                sl4 = pl.ds(q4 * 32, 32)
                pltpu.make_async_copy(hl_hbm.at[IDX[jj].at[0, sl4]],
                                      A[g].at[sl4], SG[g]).wait()
                pltpu.make_async_copy(hr_hbm.at[IDX[jj].at[1, sl4]],
                                      B[g].at[sl4], SG[g]).wait()

            def w3():
                pltpu.make_async_copy(
                    S[ng], acc_sh.at[IDX[pj].at[1]], SS[ng]).wait()

            if jj == 0:
                pl.when(step > 0)(w3)
            else:
                w3()
            pltpu.async_copy(idx_hbm.at[cid + 3], IDX[pj], SI[pj])
            compute(cid, A[g], B[g], S[g])
            pltpu.async_copy(S[g], acc_sh.at[IDX[jj].at[1]], SS[g], add=True)
        return carry

    lax.fori_loop(0, CPW // 4, step_body, 0)
    for q4 in range(4):
        sl4 = pl.ds(q4 * 32, 32)
        pltpu.make_async_copy(hl_hbm.at[IDX[0].at[0, sl4]],
                              A[0].at[sl4], SG[0]).wait()
        pltpu.make_async_copy(hr_hbm.at[IDX[0].at[1, sl4]],
                              B[0].at[sl4], SG[0]).wait()
    pltpu.make_async_copy(idx_hbm.at[0], IDX[1], SI[1]).wait()
    pltpu.make_async_copy(idx_hbm.at[0], IDX[2], SI[2]).wait()
    pltpu.make_async_copy(S[1], acc_sh.at[IDX[3].at[1]], SS[1]).wait()
    plsc.subcore_barrier()

    def dchunk(k, carry):
        cid = s + k * NS

        @pl.when(cid < NRC)
        def _():
            sl = pl.ds(cid * RCH, RCH)
            pltpu.sync_copy(acc_sh.at[sl], out_hbm.at[c, sl])

        return carry

    lax.fori_loop(0, KPS, dchunk, 0)


def _sc_edge(hl, hr, idx_all, att):
    att = lax.reduce_precision(att, exponent_bits=8, mantissa_bits=7)
    att = jnp.broadcast_to(att[:, None], (GDIM, 16))
    mesh = plsc.VectorSubcoreMesh(core_axis_name="c", subcore_axis_name="s")
    fn = pl.kernel(
        _sc_edge_body,
        mesh=mesh,
        out_type=jax.ShapeDtypeStruct((NC, N_NODES, ACCW), jnp.float32),
        scratch_types=(
            [pltpu.VMEM((2, CHUNK), jnp.int32)] * 4 +
            [pltpu.VMEM((CHUNK, GDIM), jnp.float32)] * 4 +
            [pltpu.VMEM((CHUNK, ACCW), jnp.float32)] * 2 +
            [pltpu.VMEM((GDIM, 16), jnp.float32),
             pltpu.VMEM((RCH, ACCW), jnp.float32),
             pltpu.VMEM_SHARED((N_NODES, ACCW), jnp.float32)] +
            [pltpu.SemaphoreType.DMA] * 8
        ),
        compiler_params=pltpu.CompilerParams(
            needs_layout_passes=False, use_tc_tiling_on_sc=False,
            disable_bounds_checks=True),
    )
    return fn(hl, hr, idx_all, att)


def _tc_in_body(x_ref, wl_ref, wr_ref, hl_ref, hr_ref):
    x = x_ref[...]
    hl_ref[...] = jnp.dot(x, wl_ref[...], preferred_element_type=jnp.float32)
    hr_ref[...] = jnp.dot(x, wr_ref[...], preferred_element_type=jnp.float32)


def _finalize_h(acc_ref, bias_ref, gam_ref, bet_ref):
    a0 = acc_ref[0]
    a1 = acc_ref[1]
    num = a0[:, :GDIM] + a1[:, :GDIM]
    den = a0[:, GDIM:GDIM + 1] + a1[:, GDIM:GDIM + 1]
    h = num / den + bias_ref[...]
    h = jnp.maximum(h, 0.0)
    mu = jnp.mean(h, axis=0, keepdims=True)
    var = jnp.mean((h - mu) * (h - mu), axis=0, keepdims=True)
    return gam_ref[...] * (h - mu) / jnp.sqrt(var + EPS) + bet_ref[...]


def _tc_mid_body(acc_ref, bias_ref, gam_ref, bet_ref, wl_ref, wr_ref,
                 hl_ref, hr_ref):
    hbn = _finalize_h(acc_ref, bias_ref, gam_ref, bet_ref)
    hl_ref[...] = jnp.dot(hbn, wl_ref[...], preferred_element_type=jnp.float32)
    hr_ref[...] = jnp.dot(hbn, wr_ref[...], preferred_element_type=jnp.float32)


def _tc_out_body(acc_ref, bias_ref, gam_ref, bet_ref, batch_ref,
                 w1, b1, g1, be1, w2, b2, g2, be2, w3, b3, g3, be3,
                 wo, bo, out_ref):
    hbn = _finalize_h(acc_ref, bias_ref, gam_ref, bet_ref)
    gids = lax.broadcasted_iota(jnp.int32, (N_GRAPHS, N_NODES), 0)
    onehot = (gids == batch_ref[...]).astype(jnp.float32)
    g = jnp.dot(onehot, hbn, preferred_element_type=jnp.float32,
                precision=lax.Precision.HIGHEST)
    for wr, br, gr, ber in ((w1, b1, g1, be1), (w2, b2, g2, be2),
                            (w3, b3, g3, be3)):
        g = jnp.dot(g, wr[...], preferred_element_type=jnp.float32) + br[...]
        g = jnp.maximum(g, 0.0)
        mu = jnp.mean(g, axis=0, keepdims=True)
        var = jnp.mean((g - mu) * (g - mu), axis=0, keepdims=True)
        g = gr[...] * (g - mu) / jnp.sqrt(var + EPS) + ber[...]
    out_ref[...] = jnp.dot(g, wo[...], preferred_element_type=jnp.float32) + bo[...]


def _r2(v):
    return v.reshape(1, -1)


def kernel(x, edge_index, batch, params):
    loop = jnp.arange(N_NODES, dtype=edge_index.dtype)
    pad = jnp.zeros((E_PAD - E_TOT,), edge_index.dtype)
    src = jnp.concatenate([edge_index[0], loop, pad]).reshape(-1, CHUNK)
    dst = jnp.concatenate([edge_index[1], loop, pad]).reshape(-1, CHUNK)
    idx_all = jnp.concatenate(
        [jnp.stack([src, dst], axis=1),
         jnp.zeros((TOTCH - NW * CPW, 2, CHUNK), edge_index.dtype)])

    conv = params['conv']
    cbn = params['conv_bn']

    hl, hr = pl.pallas_call(
        _tc_in_body,
        out_shape=[jax.ShapeDtypeStruct((N_NODES, GDIM), jnp.float32)] * 2,
    )(x, conv[0]['Wl'], conv[0]['Wr'])

    for l in range(2):
        acc = _sc_edge(hl, hr, idx_all, conv[l]['att'])
        hl, hr = pl.pallas_call(
            _tc_mid_body,
            out_shape=[jax.ShapeDtypeStruct((N_NODES, GDIM), jnp.float32)] * 2,
        )(acc, _r2(conv[l]['bias']), _r2(cbn[l]['gamma']), _r2(cbn[l]['beta']),
          conv[l + 1]['Wl'], conv[l + 1]['Wr'])

    acc = _sc_edge(hl, hr, idx_all, conv[2]['att'])
    dense = params['dense']
    dbn = params['dense_bn']
    dargs = []
    for l in range(3):
        dargs += [dense[l]['W'], _r2(dense[l]['b']),
                  _r2(dbn[l]['gamma']), _r2(dbn[l]['beta'])]
    y = pl.pallas_call(
        _tc_out_body,
        out_shape=jax.ShapeDtypeStruct((N_GRAPHS, 1), jnp.float32),
    )(acc, _r2(conv[2]['bias']), _r2(cbn[2]['gamma']), _r2(cbn[2]['beta']),
      batch.reshape(1, -1), *dargs, params['out']['W'], _r2(params['out']['b']))
    return y
